# baseline (speedup 1.0000x reference)
import jax
import jax.numpy as jnp
from jax.experimental import pallas as pl
from functools import partial

_NB = 1000


def _dense_stage_kernel(h_ref, wself_ref, wneigh_ref, wg_ref, wa_ref,
                        al_ref, ar_ref,
                        a_ref, b_ref, c_ref, hg_ref, el_ref, er_ref):
    h = h_ref[...]
    a_ref[...] = jnp.dot(h, wself_ref[...], preferred_element_type=jnp.float32)
    b_ref[...] = jnp.dot(h, wneigh_ref[...], preferred_element_type=jnp.float32)
    c_ref[...] = jnp.dot(h, wg_ref[...], preferred_element_type=jnp.float32)
    hg = jnp.dot(h, wa_ref[...], preferred_element_type=jnp.float32)
    hg_ref[...] = hg
    el_ref[...] = jnp.dot(hg, al_ref[...], preferred_element_type=jnp.float32)
    er_ref[...] = jnp.dot(hg, ar_ref[...], preferred_element_type=jnp.float32)


def _dense_stage(h, Wself, Wneigh, Wg, Wa, al, ar):
    N, D = h.shape
    row = lambda i: (i, 0)
    full = lambda i: (0, 0)
    return pl.pallas_call(
        _dense_stage_kernel,
        grid=(N // _NB,),
        in_specs=[pl.BlockSpec((_NB, D), row)]
                 + [pl.BlockSpec((D, D), full)] * 4
                 + [pl.BlockSpec((D, 1), full)] * 2,
        out_specs=[pl.BlockSpec((_NB, D), row)] * 4
                  + [pl.BlockSpec((_NB, 1), row)] * 2,
        out_shape=[jax.ShapeDtypeStruct((N, D), jnp.float32)] * 4
                  + [jax.ShapeDtypeStruct((N, 1), jnp.float32)] * 2,
    )(h, Wself, Wneigh, Wg, Wa, al.reshape(D, 1), ar.reshape(D, 1))


def _edge_lrelu_kernel(a_ref, b_ref, o_ref):
    z = a_ref[...] + b_ref[...]
    o_ref[...] = jnp.where(z > 0, z, 0.2 * z)


def _edge_exp_kernel(e_ref, m_ref, o_ref):
    o_ref[...] = jnp.exp(e_ref[...] - m_ref[...])


def _edge_div_kernel(ee_ref, den_ref, o_ref):
    o_ref[...] = ee_ref[...] / jnp.maximum(den_ref[...], 1e-9)


def _edge_call(fn, *args):
    E = args[0].shape[0]
    shp = (E // 128, 128)
    args2 = [a.reshape(shp) for a in args]
    out = pl.pallas_call(
        fn, out_shape=jax.ShapeDtypeStruct(shp, jnp.float32),
    )(*args2)
    return out.reshape(E)


def _combine_kernel(a_ref, sn_ref, indeg_ref, s2_ref, nd_ref, s3_ref,
                    bsage_ref, bg_ref, bga_ref, w3_ref, gamma_ref, beta_ref,
                    o_ref, *, apply_bn):
    x1 = a_ref[...] + sn_ref[...] / jnp.maximum(indeg_ref[...], 1.0) \
        + bsage_ref[...]
    x2 = s2_ref[...] * nd_ref[...] + bg_ref[...]
    x3 = s3_ref[...] + bga_ref[...]
    w3v = w3_ref[...]
    w = jax.nn.softmax(jnp.where(w3v > 0, w3v, 0.01 * w3v), axis=0)
    y = w[0:1, :] * x1 + w[1:2, :] * x2 + w[2:3, :] * x3
    if apply_bn:
        y = gamma_ref[...] * (y * (1.0 / jnp.sqrt(1.0 + 1e-05))) + beta_ref[...]
        y = jnp.maximum(y, 0.0)
    o_ref[...] = y


def _combine(A, Sn, in_deg, S2, nd, S3, bsage, bg, bga, w3, gamma, beta,
             apply_bn):
    N, D = A.shape
    row = lambda i: (i, 0)
    full = lambda i: (0, 0)
    b1 = lambda v: v.reshape(1, D)
    return pl.pallas_call(
        partial(_combine_kernel, apply_bn=apply_bn),
        grid=(N // _NB,),
        in_specs=[pl.BlockSpec((_NB, D), row),
                  pl.BlockSpec((_NB, D), row),
                  pl.BlockSpec((_NB, 1), row),
                  pl.BlockSpec((_NB, D), row),
                  pl.BlockSpec((_NB, 1), row),
                  pl.BlockSpec((_NB, D), row),
                  pl.BlockSpec((1, D), full),
                  pl.BlockSpec((1, D), full),
                  pl.BlockSpec((1, D), full),
                  pl.BlockSpec((3, D), full),
                  pl.BlockSpec((1, D), full),
                  pl.BlockSpec((1, D), full)],
        out_specs=pl.BlockSpec((_NB, D), row),
        out_shape=jax.ShapeDtypeStruct((N, D), jnp.float32),
    )(A, Sn, in_deg.reshape(N, 1), S2, nd.reshape(N, 1), S3,
      b1(bsage), b1(bg), b1(bga),
      jnp.broadcast_to(w3.reshape(3, 1), (3, D)), b1(gamma), b1(beta))


def _conet_layer(h, src, dst, in_deg, nd, ns_src, Wself, Wneigh, bsage,
                 Wg, bg, Wa, al, ar, bga, w3, gamma, beta, apply_bn):
    N, D = h.shape
    A, B, C, Hg, el, er = _dense_stage(h, Wself, Wneigh, Wg, Wa, al, ar)
    el = el[:, 0]
    er = er[:, 0]
    Sn = jax.ops.segment_sum(B[src], dst, num_segments=N)
    S2 = jax.ops.segment_sum(C[src] * ns_src[:, None], dst, num_segments=N)
    e = _edge_call(_edge_lrelu_kernel, el[src], er[dst])
    emax = jax.ops.segment_max(e, dst, num_segments=N)
    emax = jnp.where(jnp.isfinite(emax), emax, 0.0)
    ee = _edge_call(_edge_exp_kernel, e, emax[dst])
    den = jax.ops.segment_sum(ee, dst, num_segments=N)
    alpha = _edge_call(_edge_div_kernel, ee, den[dst])
    S3 = jax.ops.segment_sum(alpha[:, None] * Hg[src], dst, num_segments=N)
    return _combine(A, Sn, in_deg, S2, nd, S3, bsage, bg, bga, w3,
                    gamma, beta, apply_bn)


def kernel(x, edge_index, l1_Wself, l1_Wneigh, l1_bsage, l1_Wg, l1_bg, l1_Wa,
           l1_al, l1_ar, l1_bga, l1_w3, bn_gamma, bn_beta, l2_Wself,
           l2_Wneigh, l2_bsage, l2_Wg, l2_bg, l2_Wa, l2_al, l2_ar, l2_bga,
           l2_w3):
    src = edge_index[0]
    dst = edge_index[1]
    N = x.shape[0]
    E = src.shape[0]
    ones = jnp.ones((E,), dtype=x.dtype)
    in_deg = jax.ops.segment_sum(ones, dst, num_segments=N)
    out_deg = jax.ops.segment_sum(ones, src, num_segments=N)
    ns = jnp.where(out_deg > 0, 1.0 / jnp.sqrt(jnp.clip(out_deg, 1.0)), 0.0)
    nd = jnp.where(in_deg > 0, 1.0 / jnp.sqrt(jnp.clip(in_deg, 1.0)), 0.0)
    ns_src = ns[src]
    h = _conet_layer(x, src, dst, in_deg, nd, ns_src,
                     l1_Wself, l1_Wneigh, l1_bsage, l1_Wg, l1_bg, l1_Wa,
                     l1_al, l1_ar, l1_bga, l1_w3, bn_gamma, bn_beta,
                     apply_bn=True)
    out = _conet_layer(h, src, dst, in_deg, nd, ns_src,
                       l2_Wself, l2_Wneigh, l2_bsage, l2_Wg, l2_bg, l2_Wa,
                       l2_al, l2_ar, l2_bga, l2_w3, bn_gamma, bn_beta,
                       apply_bn=False)
    return out


# speedup vs baseline: 1.0339x; 1.0339x over previous
import jax
import jax.numpy as jnp
from jax.experimental import pallas as pl
from functools import partial

_NB = 1000
_EB = 2000


def _dense_stage_kernel(h_ref, wself_ref, wneigh_ref, wg_ref, wa_ref,
                        al_ref, ar_ref, ns_ref, w0_ref, w1_ref, w2_ref,
                        a_ref, p_ref, el_ref, er_ref, *, D):
    h = h_ref[...]
    a_ref[...] = jnp.dot(h, wself_ref[...], preferred_element_type=jnp.float32)
    p_ref[:, 0:D] = w0_ref[...] * jnp.dot(
        h, wneigh_ref[...], preferred_element_type=jnp.float32)
    p_ref[:, D:2 * D] = (w1_ref[...] * ns_ref[...]) * jnp.dot(
        h, wg_ref[...], preferred_element_type=jnp.float32)
    hg = jnp.dot(h, wa_ref[...], preferred_element_type=jnp.float32)
    p_ref[:, 2 * D:3 * D] = w2_ref[...] * hg
    el_ref[...] = jnp.dot(hg, al_ref[...], preferred_element_type=jnp.float32)
    er_ref[...] = jnp.dot(hg, ar_ref[...], preferred_element_type=jnp.float32)


def _dense_stage(h, Wself, Wneigh, Wg, Wa, al, ar, ns, w):
    N, D = h.shape
    row = lambda i: (i, 0)
    full = lambda i: (0, 0)
    wrow = lambda s: jnp.broadcast_to(s, (1, D))
    return pl.pallas_call(
        partial(_dense_stage_kernel, D=D),
        grid=(N // _NB,),
        in_specs=[pl.BlockSpec((_NB, D), row)]
                 + [pl.BlockSpec((D, D), full)] * 4
                 + [pl.BlockSpec((D, 1), full)] * 2
                 + [pl.BlockSpec((_NB, 1), row)]
                 + [pl.BlockSpec((1, D), full)] * 3,
        out_specs=[pl.BlockSpec((_NB, D), row),
                   pl.BlockSpec((_NB, 3 * D), row),
                   pl.BlockSpec((_NB, 1), row),
                   pl.BlockSpec((_NB, 1), row)],
        out_shape=[jax.ShapeDtypeStruct((N, D), jnp.float32),
                   jax.ShapeDtypeStruct((N, 3 * D), jnp.float32),
                   jax.ShapeDtypeStruct((N, 1), jnp.float32),
                   jax.ShapeDtypeStruct((N, 1), jnp.float32)],
    )(h, Wself, Wneigh, Wg, Wa, al.reshape(D, 1), ar.reshape(D, 1),
      ns.reshape(N, 1), wrow(w[0]), wrow(w[1]), wrow(w[2]))


def _edge_lrelu_kernel(a_ref, b_ref, o_ref):
    z = a_ref[...] + b_ref[...]
    o_ref[...] = jnp.where(z > 0, z, 0.2 * z)


def _edge_exp_kernel(e_ref, m_ref, o_ref):
    o_ref[...] = jnp.exp(e_ref[...] - m_ref[...])


def _edge_call(fn, *args):
    E = args[0].shape[0]
    shp = (E // 128, 128)
    out = pl.pallas_call(
        fn, out_shape=jax.ShapeDtypeStruct(shp, jnp.float32),
    )(*[a.reshape(shp) for a in args])
    return out.reshape(E)


def _update_kernel(pg_ref, indeg_ref, nd_ref, ee_ref, den_ref, o_ref, *, D):
    c1 = 1.0 / jnp.maximum(indeg_ref[...], 1.0)
    c2 = nd_ref[...]
    c3 = ee_ref[...] / jnp.maximum(den_ref[...], 1e-9)
    pg = pg_ref[...]
    o_ref[...] = c1 * pg[:, 0:D] + c2 * pg[:, D:2 * D] + c3 * pg[:, 2 * D:3 * D]


def _update(Pg, indeg_d, nd_d, ee, den_d):
    E = Pg.shape[0]
    D = Pg.shape[1] // 3
    row = lambda i: (i, 0)
    col = lambda v: v.reshape(E, 1)
    return pl.pallas_call(
        partial(_update_kernel, D=D),
        grid=(E // _EB,),
        in_specs=[pl.BlockSpec((_EB, 3 * D), row)]
                 + [pl.BlockSpec((_EB, 1), row)] * 4,
        out_specs=pl.BlockSpec((_EB, D), row),
        out_shape=jax.ShapeDtypeStruct((E, D), jnp.float32),
    )(Pg, col(indeg_d), col(nd_d), col(ee), col(den_d))


def _combine_kernel(a_ref, s_ref, w0_ref, w1_ref, w2_ref, bsage_ref, bg_ref,
                    bga_ref, gamma_ref, beta_ref, o_ref, *, apply_bn):
    y = (w0_ref[...] * a_ref[...] + s_ref[...]
         + w0_ref[...] * bsage_ref[...] + w1_ref[...] * bg_ref[...]
         + w2_ref[...] * bga_ref[...])
    if apply_bn:
        y = gamma_ref[...] * (y * (1.0 / jnp.sqrt(1.0 + 1e-05))) + beta_ref[...]
        y = jnp.maximum(y, 0.0)
    o_ref[...] = y


def _combine(A, S, w, bsage, bg, bga, gamma, beta, apply_bn):
    N, D = A.shape
    row = lambda i: (i, 0)
    full = lambda i: (0, 0)
    wrow = lambda s: jnp.broadcast_to(s, (1, D))
    b1 = lambda v: v.reshape(1, D)
    return pl.pallas_call(
        partial(_combine_kernel, apply_bn=apply_bn),
        grid=(N // _NB,),
        in_specs=[pl.BlockSpec((_NB, D), row), pl.BlockSpec((_NB, D), row)]
                 + [pl.BlockSpec((1, D), full)] * 8,
        out_specs=pl.BlockSpec((_NB, D), row),
        out_shape=jax.ShapeDtypeStruct((N, D), jnp.float32),
    )(A, S, wrow(w[0]), wrow(w[1]), wrow(w[2]),
      b1(bsage), b1(bg), b1(bga), b1(gamma), b1(beta))


def _conet_layer(h, src, dst, in_deg, nd, ns, Wself, Wneigh, bsage,
                 Wg, bg, Wa, al, ar, bga, w3, gamma, beta, apply_bn):
    N, D = h.shape
    w = jax.nn.softmax(jax.nn.leaky_relu(w3, 0.01))
    A, P, el, er = _dense_stage(h, Wself, Wneigh, Wg, Wa, al, ar, ns, w)
    el = el[:, 0]
    er = er[:, 0]
    e = _edge_call(_edge_lrelu_kernel, el[src], er[dst])
    emax = jax.ops.segment_max(e, dst, num_segments=N)
    emax = jnp.where(jnp.isfinite(emax), emax, 0.0)
    ee = _edge_call(_edge_exp_kernel, e, emax[dst])
    den = jax.ops.segment_sum(ee, dst, num_segments=N)
    upd = _update(P[src], in_deg[dst], nd[dst], ee, den[dst])
    S = jax.ops.segment_sum(upd, dst, num_segments=N)
    return _combine(A, S, w, bsage, bg, bga, gamma, beta, apply_bn)


def kernel(x, edge_index, l1_Wself, l1_Wneigh, l1_bsage, l1_Wg, l1_bg, l1_Wa,
           l1_al, l1_ar, l1_bga, l1_w3, bn_gamma, bn_beta, l2_Wself,
           l2_Wneigh, l2_bsage, l2_Wg, l2_bg, l2_Wa, l2_al, l2_ar, l2_bga,
           l2_w3):
    src = edge_index[0]
    dst = edge_index[1]
    N = x.shape[0]
    E = src.shape[0]
    ones = jnp.ones((E,), dtype=x.dtype)
    in_deg = jax.ops.segment_sum(ones, dst, num_segments=N)
    out_deg = jax.ops.segment_sum(ones, src, num_segments=N)
    ns = jnp.where(out_deg > 0, 1.0 / jnp.sqrt(jnp.clip(out_deg, 1.0)), 0.0)
    nd = jnp.where(in_deg > 0, 1.0 / jnp.sqrt(jnp.clip(in_deg, 1.0)), 0.0)
    h = _conet_layer(x, src, dst, in_deg, nd, ns,
                     l1_Wself, l1_Wneigh, l1_bsage, l1_Wg, l1_bg, l1_Wa,
                     l1_al, l1_ar, l1_bga, l1_w3, bn_gamma, bn_beta,
                     apply_bn=True)
    out = _conet_layer(h, src, dst, in_deg, nd, ns,
                       l2_Wself, l2_Wneigh, l2_bsage, l2_Wg, l2_bg, l2_Wa,
                       l2_al, l2_ar, l2_bga, l2_w3, bn_gamma, bn_beta,
                       apply_bn=False)
    return out


# speedup vs baseline: 1.0804x; 1.0449x over previous
import jax
import jax.numpy as jnp
from jax.experimental import pallas as pl
from functools import partial

_NB = 1000
_EB = 2000


def _dense_stage_kernel(h_ref, wself_ref, wneigh_ref, wg_ref, wa_ref,
                        al_ref, ar_ref, ns_ref, w0_ref, w1_ref, w2_ref,
                        a_ref, p_ref, el_ref, er_ref, *, D):
    h = h_ref[...]
    a_ref[...] = jnp.dot(h, wself_ref[...], preferred_element_type=jnp.float32)
    p_ref[:, 0:D] = w0_ref[...] * jnp.dot(
        h, wneigh_ref[...], preferred_element_type=jnp.float32)
    p_ref[:, D:2 * D] = (w1_ref[...] * ns_ref[...]) * jnp.dot(
        h, wg_ref[...], preferred_element_type=jnp.float32)
    hg = jnp.dot(h, wa_ref[...], preferred_element_type=jnp.float32)
    p_ref[:, 2 * D:3 * D] = w2_ref[...] * hg
    el_ref[...] = jnp.dot(hg, al_ref[...], preferred_element_type=jnp.float32)
    er_ref[...] = jnp.dot(hg, ar_ref[...], preferred_element_type=jnp.float32)


def _dense_stage(h, Wself, Wneigh, Wg, Wa, al, ar, ns, w):
    N, D = h.shape
    row = lambda i: (i, 0)
    full = lambda i: (0, 0)
    wrow = lambda s: jnp.broadcast_to(s, (1, D))
    return pl.pallas_call(
        partial(_dense_stage_kernel, D=D),
        grid=(N // _NB,),
        in_specs=[pl.BlockSpec((_NB, D), row)]
                 + [pl.BlockSpec((D, D), full)] * 4
                 + [pl.BlockSpec((D, 1), full)] * 2
                 + [pl.BlockSpec((_NB, 1), row)]
                 + [pl.BlockSpec((1, D), full)] * 3,
        out_specs=[pl.BlockSpec((_NB, D), row),
                   pl.BlockSpec((_NB, 3 * D), row),
                   pl.BlockSpec((_NB, 1), row),
                   pl.BlockSpec((_NB, 1), row)],
        out_shape=[jax.ShapeDtypeStruct((N, D), jnp.float32),
                   jax.ShapeDtypeStruct((N, 3 * D), jnp.float32),
                   jax.ShapeDtypeStruct((N, 1), jnp.float32),
                   jax.ShapeDtypeStruct((N, 1), jnp.float32)],
    )(h, Wself, Wneigh, Wg, Wa, al.reshape(D, 1), ar.reshape(D, 1),
      ns.reshape(N, 1), wrow(w[0]), wrow(w[1]), wrow(w[2]))


def _edge_lrelu_kernel(a_ref, b_ref, o_ref):
    z = a_ref[...] + b_ref[...]
    o_ref[...] = jnp.where(z > 0, z, 0.2 * z)


def _edge_exp_kernel(e_ref, m_ref, o_ref):
    o_ref[...] = jnp.exp(e_ref[...] - m_ref[...])


def _edge_call(fn, *args):
    E = args[0].shape[0]
    shp = (E // 128, 128)
    out = pl.pallas_call(
        fn, out_shape=jax.ShapeDtypeStruct(shp, jnp.float32),
    )(*[a.reshape(shp) for a in args])
    return out.reshape(E)


def _update_kernel(pg_ref, indeg_ref, nd_ref, ee_ref, den_ref, o_ref, *, D):
    c1 = 1.0 / jnp.maximum(indeg_ref[...], 1.0)
    c2 = nd_ref[...]
    c3 = ee_ref[...] / jnp.maximum(den_ref[...], 1e-9)
    pg = pg_ref[...]
    o_ref[...] = c1 * pg[:, 0:D] + c2 * pg[:, D:2 * D] + c3 * pg[:, 2 * D:3 * D]


def _update(Pg, indeg_d, nd_d, ee, den_d):
    E = Pg.shape[0]
    D = Pg.shape[1] // 3
    row = lambda i: (i, 0)
    col = lambda v: v.reshape(E, 1)
    return pl.pallas_call(
        partial(_update_kernel, D=D),
        grid=(E // _EB,),
        in_specs=[pl.BlockSpec((_EB, 3 * D), row)]
                 + [pl.BlockSpec((_EB, 1), row)] * 4,
        out_specs=pl.BlockSpec((_EB, D), row),
        out_shape=jax.ShapeDtypeStruct((E, D), jnp.float32),
    )(Pg, col(indeg_d), col(nd_d), col(ee), col(den_d))


def _combine_kernel(a_ref, s_ref, w0_ref, w1_ref, w2_ref, bsage_ref, bg_ref,
                    bga_ref, gamma_ref, beta_ref, o_ref, *, apply_bn):
    y = (w0_ref[...] * a_ref[...] + s_ref[...]
         + w0_ref[...] * bsage_ref[...] + w1_ref[...] * bg_ref[...]
         + w2_ref[...] * bga_ref[...])
    if apply_bn:
        y = gamma_ref[...] * (y * (1.0 / jnp.sqrt(1.0 + 1e-05))) + beta_ref[...]
        y = jnp.maximum(y, 0.0)
    o_ref[...] = y


def _combine(A, S, w, bsage, bg, bga, gamma, beta, apply_bn):
    N, D = A.shape
    row = lambda i: (i, 0)
    full = lambda i: (0, 0)
    wrow = lambda s: jnp.broadcast_to(s, (1, D))
    b1 = lambda v: v.reshape(1, D)
    return pl.pallas_call(
        partial(_combine_kernel, apply_bn=apply_bn),
        grid=(N // _NB,),
        in_specs=[pl.BlockSpec((_NB, D), row), pl.BlockSpec((_NB, D), row)]
                 + [pl.BlockSpec((1, D), full)] * 8,
        out_specs=pl.BlockSpec((_NB, D), row),
        out_shape=jax.ShapeDtypeStruct((N, D), jnp.float32),
    )(A, S, wrow(w[0]), wrow(w[1]), wrow(w[2]),
      b1(bsage), b1(bg), b1(bga), b1(gamma), b1(beta))


def _conet_layer(h, src, dst, in_deg, nd, ns, Wself, Wneigh, bsage,
                 Wg, bg, Wa, al, ar, bga, w3, gamma, beta, apply_bn):
    N, D = h.shape
    w = jax.nn.softmax(jax.nn.leaky_relu(w3, 0.01))
    A, P, el, er = _dense_stage(h, Wself, Wneigh, Wg, Wa, al, ar, ns, w)
    el = el[:, 0]
    er = er[:, 0]
    e = _edge_call(_edge_lrelu_kernel, el[src], er[dst])
    emax = jax.ops.segment_max(e, dst, num_segments=N,
                               indices_are_sorted=True)
    emax = jnp.where(jnp.isfinite(emax), emax, 0.0)
    ee = _edge_call(_edge_exp_kernel, e, emax[dst])
    den = jax.ops.segment_sum(ee, dst, num_segments=N,
                              indices_are_sorted=True)
    upd = _update(P[src], in_deg[dst], nd[dst], ee, den[dst])
    S = jax.ops.segment_sum(upd, dst, num_segments=N,
                            indices_are_sorted=True)
    return _combine(A, S, w, bsage, bg, bga, gamma, beta, apply_bn)


def kernel(x, edge_index, l1_Wself, l1_Wneigh, l1_bsage, l1_Wg, l1_bg, l1_Wa,
           l1_al, l1_ar, l1_bga, l1_w3, bn_gamma, bn_beta, l2_Wself,
           l2_Wneigh, l2_bsage, l2_Wg, l2_bg, l2_Wa, l2_al, l2_ar, l2_bga,
           l2_w3):
    N = x.shape[0]
    E = edge_index.shape[1]
    perm = jnp.argsort(edge_index[1])
    dst = edge_index[1][perm]
    src = edge_index[0][perm]
    ones = jnp.ones((E,), dtype=x.dtype)
    in_deg = jax.ops.segment_sum(ones, dst, num_segments=N,
                                 indices_are_sorted=True)
    out_deg = jax.ops.segment_sum(ones, src, num_segments=N)
    ns = jnp.where(out_deg > 0, 1.0 / jnp.sqrt(jnp.clip(out_deg, 1.0)), 0.0)
    nd = jnp.where(in_deg > 0, 1.0 / jnp.sqrt(jnp.clip(in_deg, 1.0)), 0.0)
    h = _conet_layer(x, src, dst, in_deg, nd, ns,
                     l1_Wself, l1_Wneigh, l1_bsage, l1_Wg, l1_bg, l1_Wa,
                     l1_al, l1_ar, l1_bga, l1_w3, bn_gamma, bn_beta,
                     apply_bn=True)
    out = _conet_layer(h, src, dst, in_deg, nd, ns,
                       l2_Wself, l2_Wneigh, l2_bsage, l2_Wg, l2_bg, l2_Wa,
                       l2_al, l2_ar, l2_bga, l2_w3, bn_gamma, bn_beta,
                       apply_bn=False)
    return out
